# baseline (speedup 1.0000x reference)
import functools

import jax
import jax.numpy as jnp
from jax import lax
from jax.experimental import pallas as pl
from jax.experimental.pallas import tpu as pltpu
from jax.experimental.pallas import tpu_sc as plsc

VOCAB = 100000
EMB = 64
U1 = 128
U2 = 512
B = 1024
T = 50
VT = 4096
NVT = (VOCAB + VT - 1) // VT


EP = 128
CHUNK = 80
NCHUNK = 20
HALF = 10


@functools.lru_cache(maxsize=1)
def _make_sc_gather():
    nc, ns = 2, 16
    nw = nc * ns
    tb = T * B
    bpw = tb // nw

    mesh = plsc.VectorSubcoreMesh(core_axis_name="c", subcore_axis_name="s")

    @functools.partial(
        pl.kernel,
        out_type=jax.ShapeDtypeStruct((nw * 2, HALF, CHUNK, EP), jnp.float32),
        mesh=mesh,
        scratch_types=[
            pltpu.VMEM((bpw,), jnp.int32),
            pltpu.VMEM((HALF, CHUNK, EP), jnp.float32),
            pltpu.SemaphoreType.DMA,
        ],
    )
    def gather_kernel(emb_hbm, idx_hbm, out_hbm, idx_v, rows_v, sem):
        wid = lax.axis_index("s") * nc + lax.axis_index("c")
        base = wid * bpw
        pltpu.sync_copy(idx_hbm.at[pl.ds(base, bpw)], idx_v)
        for half in range(2):
            copies = []
            for k in range(HALF):
                kk = half * HALF + k
                copies.append(pltpu.async_copy(
                    emb_hbm.at[idx_v.at[pl.ds(kk * CHUNK, CHUNK)]],
                    rows_v.at[k], sem))
            for c in copies:
                c.wait()
            pltpu.sync_copy(rows_v, out_hbm.at[wid * 2 + half])

    return gather_kernel


def _lstm_body(e_ref, w1_ref, r1_ref, b1_ref, w2_ref, r2_ref, b2_ref,
               out_ref, h1_ref, c1_ref, h2_ref, c2_ref):
    h1_ref[...] = jnp.zeros((B, U1), jnp.float32)
    c1_ref[...] = jnp.zeros((B, U1), jnp.float32)
    h2_ref[...] = jnp.zeros((B, U2), jnp.float32)
    c2_ref[...] = jnp.zeros((B, U2), jnp.float32)

    def gates(z, units):
        i = jax.nn.sigmoid(z[:, :units])
        f = jax.nn.sigmoid(z[:, units:2 * units])
        g = jnp.tanh(z[:, 2 * units:3 * units])
        o = jax.nn.sigmoid(z[:, 3 * units:])
        return i, f, g, o

    def step(t, _):
        xt = e_ref[t].astype(jnp.bfloat16)
        z1 = (jnp.dot(xt, w1_ref[...], preferred_element_type=jnp.float32)
              + jnp.dot(h1_ref[...].astype(jnp.bfloat16), r1_ref[...],
                        preferred_element_type=jnp.float32)
              + b1_ref[...])
        i1, f1, g1, o1 = gates(z1, U1)
        c1 = f1 * c1_ref[...] + i1 * g1
        h1 = o1 * jnp.tanh(c1)
        c1_ref[...] = c1
        h1_ref[...] = h1
        z2 = (jnp.dot(h1.astype(jnp.bfloat16), w2_ref[...],
                      preferred_element_type=jnp.float32)
              + jnp.dot(h2_ref[...].astype(jnp.bfloat16), r2_ref[...],
                        preferred_element_type=jnp.float32)
              + b2_ref[...])
        i2, f2, g2, o2 = gates(z2, U2)
        c2 = f2 * c2_ref[...] + i2 * g2
        h2_ref[...] = o2 * jnp.tanh(c2)
        c2_ref[...] = c2
        return 0

    lax.fori_loop(0, T, step, 0)
    out_ref[...] = h2_ref[...]


def _lstm(e, W1, R1, b1, W2, R2, b2):
    return pl.pallas_call(
        _lstm_body,
        out_shape=jax.ShapeDtypeStruct((B, U2), jnp.float32),
        scratch_shapes=[
            pltpu.VMEM((B, U1), jnp.float32),
            pltpu.VMEM((B, U1), jnp.float32),
            pltpu.VMEM((B, U2), jnp.float32),
            pltpu.VMEM((B, U2), jnp.float32),
        ],
    )(e, W1.astype(jnp.bfloat16), R1.astype(jnp.bfloat16),
      b1.reshape(1, 4 * U1), W2.astype(jnp.bfloat16),
      R2.astype(jnp.bfloat16), b2.reshape(1, 4 * U2))


def _logits_tile(h2_ref, wd_ref, bd_ref, j):
    l = (jnp.dot(h2_ref[...], wd_ref[...].astype(jnp.bfloat16),
                 preferred_element_type=jnp.float32)
         + bd_ref[...])
    col = j * VT + lax.broadcasted_iota(jnp.int32, (B, VT), 1)
    return jnp.where(col < VOCAB, l, -jnp.inf)


def _stats_body(h2_ref, wd_ref, bd_ref, m_ref, s_ref):
    j = pl.program_id(0)
    l = _logits_tile(h2_ref, wd_ref, bd_ref, j)
    bm = jnp.max(l, axis=1, keepdims=True)

    @pl.when(j == 0)
    def _():
        m_ref[...] = bm
        s_ref[...] = jnp.sum(jnp.exp(l - bm), axis=1, keepdims=True)

    @pl.when(j > 0)
    def _():
        m_old = m_ref[...]
        m_new = jnp.maximum(m_old, bm)
        s_ref[...] = (s_ref[...] * jnp.exp(m_old - m_new)
                      + jnp.sum(jnp.exp(l - m_new), axis=1, keepdims=True))
        m_ref[...] = m_new


def _norm_body(h2_ref, wd_ref, bd_ref, m_ref, s_ref, out_ref):
    j = pl.program_id(0)
    l = _logits_tile(h2_ref, wd_ref, bd_ref, j)
    out_ref[...] = jnp.exp(l - m_ref[...]) / s_ref[...]


def _dense_softmax(h2, Wd, bd):
    h2 = h2.astype(jnp.bfloat16)
    bd2 = bd.reshape(1, VOCAB)
    m, s = pl.pallas_call(
        _stats_body,
        grid=(NVT,),
        in_specs=[
            pl.BlockSpec((B, U2), lambda j: (0, 0)),
            pl.BlockSpec((U2, VT), lambda j: (0, j)),
            pl.BlockSpec((1, VT), lambda j: (0, j)),
        ],
        out_specs=[
            pl.BlockSpec((B, 1), lambda j: (0, 0)),
            pl.BlockSpec((B, 1), lambda j: (0, 0)),
        ],
        out_shape=[
            jax.ShapeDtypeStruct((B, 1), jnp.float32),
            jax.ShapeDtypeStruct((B, 1), jnp.float32),
        ],
    )(h2, Wd, bd2)
    return pl.pallas_call(
        _norm_body,
        grid=(NVT,),
        in_specs=[
            pl.BlockSpec((B, U2), lambda j: (0, 0)),
            pl.BlockSpec((U2, VT), lambda j: (0, j)),
            pl.BlockSpec((1, VT), lambda j: (0, j)),
            pl.BlockSpec((B, 1), lambda j: (0, 0)),
            pl.BlockSpec((B, 1), lambda j: (0, 0)),
        ],
        out_specs=pl.BlockSpec((B, VT), lambda j: (0, j)),
        out_shape=jax.ShapeDtypeStruct((B, VOCAB), jnp.float32),
    )(h2, Wd, bd2, m, s)


def kernel(x, emb, W1, R1, b1, W2, R2, b2, Wd, bd):
    idx = x.astype(jnp.int32).T.reshape(T * B)
    emb_p = jnp.pad(emb, ((0, 0), (0, EP - EMB)))
    W1_p = jnp.pad(W1, ((0, EP - EMB), (0, 0)))
    e = _make_sc_gather()(emb_p, idx).reshape(T, B, EP)
    h2 = _lstm(e, W1_p, R1, b1, W2, R2, b2)
    return _dense_softmax(h2, Wd, bd)


# speedup vs baseline: 1.5779x; 1.5779x over previous
import functools

import jax
import jax.numpy as jnp
from jax import lax
from jax.experimental import pallas as pl
from jax.experimental.pallas import tpu as pltpu
from jax.experimental.pallas import tpu_sc as plsc

VOCAB = 100000
EMB = 64
U1 = 128
U2 = 512
B = 1024
T = 50
VT = 2048
NVT = (VOCAB + VT - 1) // VT


EP = 128
CHUNK = 80
NCHUNK = 20
HALF = 10


@functools.lru_cache(maxsize=1)
def _make_sc_gather():
    nc, ns = 2, 16
    nw = nc * ns
    tb = T * B
    bpw = tb // nw

    mesh = plsc.VectorSubcoreMesh(core_axis_name="c", subcore_axis_name="s")

    @functools.partial(
        pl.kernel,
        out_type=jax.ShapeDtypeStruct((nw * 2, HALF, CHUNK, EP), jnp.float32),
        mesh=mesh,
        scratch_types=[
            pltpu.VMEM((bpw,), jnp.int32),
            pltpu.VMEM((HALF, CHUNK, EP), jnp.float32),
            pltpu.SemaphoreType.DMA,
        ],
    )
    def gather_kernel(emb_hbm, idx_hbm, out_hbm, idx_v, rows_v, sem):
        wid = lax.axis_index("s") * nc + lax.axis_index("c")
        base = wid * bpw
        pltpu.sync_copy(idx_hbm.at[pl.ds(base, bpw)], idx_v)
        for half in range(2):
            copies = []
            for k in range(HALF):
                kk = half * HALF + k
                copies.append(pltpu.async_copy(
                    emb_hbm.at[idx_v.at[pl.ds(kk * CHUNK, CHUNK)]],
                    rows_v.at[k], sem))
            for c in copies:
                c.wait()
            pltpu.sync_copy(rows_v, out_hbm.at[wid * 2 + half])

    return gather_kernel


def _lstm_body(e_ref, w1_ref, r1_ref, b1_ref, w2_ref, r2_ref, b2_ref,
               out_ref, h1_ref, c1_ref, h2_ref, c2_ref):
    h1_ref[...] = jnp.zeros((B, U1), jnp.float32)
    c1_ref[...] = jnp.zeros((B, U1), jnp.float32)
    h2_ref[...] = jnp.zeros((B, U2), jnp.float32)
    c2_ref[...] = jnp.zeros((B, U2), jnp.float32)

    def gates(z, units):
        i = jax.nn.sigmoid(z[:, :units])
        f = jax.nn.sigmoid(z[:, units:2 * units])
        g = jnp.tanh(z[:, 2 * units:3 * units])
        o = jax.nn.sigmoid(z[:, 3 * units:])
        return i, f, g, o

    def step(t, _):
        xt = e_ref[t].astype(jnp.bfloat16)
        z1 = (jnp.dot(xt, w1_ref[...], preferred_element_type=jnp.float32)
              + jnp.dot(h1_ref[...].astype(jnp.bfloat16), r1_ref[...],
                        preferred_element_type=jnp.float32)
              + b1_ref[...])
        i1, f1, g1, o1 = gates(z1, U1)
        c1 = f1 * c1_ref[...] + i1 * g1
        h1 = o1 * jnp.tanh(c1)
        c1_ref[...] = c1
        h1_ref[...] = h1
        z2 = (jnp.dot(h1.astype(jnp.bfloat16), w2_ref[...],
                      preferred_element_type=jnp.float32)
              + jnp.dot(h2_ref[...].astype(jnp.bfloat16), r2_ref[...],
                        preferred_element_type=jnp.float32)
              + b2_ref[...])
        i2, f2, g2, o2 = gates(z2, U2)
        c2 = f2 * c2_ref[...] + i2 * g2
        h2_ref[...] = o2 * jnp.tanh(c2)
        c2_ref[...] = c2
        return 0

    lax.fori_loop(0, T, step, 0)
    out_ref[...] = h2_ref[...]


def _lstm(e, W1, R1, b1, W2, R2, b2):
    return pl.pallas_call(
        _lstm_body,
        out_shape=jax.ShapeDtypeStruct((B, U2), jnp.float32),
        scratch_shapes=[
            pltpu.VMEM((B, U1), jnp.float32),
            pltpu.VMEM((B, U1), jnp.float32),
            pltpu.VMEM((B, U2), jnp.float32),
            pltpu.VMEM((B, U2), jnp.float32),
        ],
    )(e, W1.astype(jnp.bfloat16), R1.astype(jnp.bfloat16),
      b1.reshape(1, 4 * U1), W2.astype(jnp.bfloat16),
      R2.astype(jnp.bfloat16), b2.reshape(1, 4 * U2))


def _logits_tile(h2t_ref, wdt_ref, bd_ref, j):
    l = (jnp.dot(wdt_ref[...].astype(jnp.bfloat16), h2t_ref[...],
                 preferred_element_type=jnp.float32)
         + bd_ref[...])
    row = j * VT + lax.broadcasted_iota(jnp.int32, (VT, B), 0)
    return jnp.where(row < VOCAB, l, -jnp.inf)


def _stats_body(h2t_ref, wdt_ref, bd_ref, m_ref, s_ref):
    j = pl.program_id(0)
    l = _logits_tile(h2t_ref, wdt_ref, bd_ref, j)
    bm = jnp.max(l, axis=0, keepdims=True)

    @pl.when(j == 0)
    def _():
        m_ref[...] = bm
        s_ref[...] = jnp.sum(jnp.exp(l - bm), axis=0, keepdims=True)

    @pl.when(j > 0)
    def _():
        m_old = m_ref[...]
        m_new = jnp.maximum(m_old, bm)
        s_ref[...] = (s_ref[...] * jnp.exp(m_old - m_new)
                      + jnp.sum(jnp.exp(l - m_new), axis=0, keepdims=True))
        m_ref[...] = m_new


def _norm_body(h2t_ref, wdt_ref, bd_ref, m_ref, s_ref, out_ref):
    j = pl.program_id(0)
    l = _logits_tile(h2t_ref, wdt_ref, bd_ref, j)
    out_ref[...] = jnp.exp(l - m_ref[...]) / s_ref[...]


def _dense_softmax(h2, Wd, bd):
    wdt = Wd.T
    h2t = h2.astype(jnp.bfloat16).T
    bd2 = bd.reshape(VOCAB, 1)
    m, s = pl.pallas_call(
        _stats_body,
        grid=(NVT,),
        in_specs=[
            pl.BlockSpec((U2, B), lambda j: (0, 0)),
            pl.BlockSpec((VT, U2), lambda j: (j, 0)),
            pl.BlockSpec((VT, 1), lambda j: (j, 0)),
        ],
        out_specs=[
            pl.BlockSpec((1, B), lambda j: (0, 0)),
            pl.BlockSpec((1, B), lambda j: (0, 0)),
        ],
        out_shape=[
            jax.ShapeDtypeStruct((1, B), jnp.float32),
            jax.ShapeDtypeStruct((1, B), jnp.float32),
        ],
    )(h2t, wdt, bd2)
    out_t = pl.pallas_call(
        _norm_body,
        grid=(NVT,),
        in_specs=[
            pl.BlockSpec((U2, B), lambda j: (0, 0)),
            pl.BlockSpec((VT, U2), lambda j: (j, 0)),
            pl.BlockSpec((VT, 1), lambda j: (j, 0)),
            pl.BlockSpec((1, B), lambda j: (0, 0)),
            pl.BlockSpec((1, B), lambda j: (0, 0)),
        ],
        out_specs=pl.BlockSpec((VT, B), lambda j: (j, 0)),
        out_shape=jax.ShapeDtypeStruct((VOCAB, B), jnp.float32),
    )(h2t, wdt, bd2, m, s)
    return out_t.T


def kernel(x, emb, W1, R1, b1, W2, R2, b2, Wd, bd):
    idx = x.astype(jnp.int32).T.reshape(T * B)
    emb_p = jnp.pad(emb, ((0, 0), (0, EP - EMB)))
    W1_p = jnp.pad(W1, ((0, EP - EMB), (0, 0)))
    e = _make_sc_gather()(emb_p, idx).reshape(T, B, EP)
    h2 = _lstm(e, W1_p, R1, b1, W2, R2, b2)
    return _dense_softmax(h2, Wd, bd)
